# baseline (speedup 1.0000x reference)
import functools

import jax
import jax.numpy as jnp
from jax import lax
from jax.experimental import pallas as pl
from jax.experimental.pallas import tpu as pltpu
from jax.experimental.pallas import tpu_sc as plsc

B, N_LOC, D, E, STEPS, NC = 8, 1250, 128, 320000, 3, 3
BN = B * N_LOC

NSC = 2
NTL = 16
NW = NSC * NTL
EPW = E // NW
K = 48
NCHF = EPW // K
REM = EPW - NCHF * K
MROW = 80
BNP = MROW * D


def _sc_aggregate_body(rel_hbm, q_hbm, label_hbm, batch_hbm, head_hbm,
                       tail_hbm, out_hbm, mout_hbm,
                       label_v, q_v, batch_v, head_v, tail_v, tail_sc,
                       tail16_v, rel_v, rows_v, mask_v, iidx_v,
                       acc_sh, macc_sh, sem_rel, sem_idx, sem_sc, sem_z):
    c = lax.axis_index("c")
    s = lax.axis_index("s")
    base_e = (c * NTL + s) * EPW

    def start_load(ci, b):
        e0 = base_e + ci * K
        pltpu.async_copy(batch_hbm.at[pl.ds(e0, K)], batch_v.at[b],
                         sem_idx.at[b])
        pltpu.async_copy(head_hbm.at[pl.ds(e0, K)], head_v.at[b],
                         sem_idx.at[b])
        pltpu.async_copy(tail_hbm.at[pl.ds(e0, K)], tail_v.at[b],
                         sem_idx.at[b])
        pltpu.async_copy(rel_hbm.at[pl.ds(e0, K), :], rel_v.at[b],
                         sem_rel.at[b])

    def wait_load(ci, b):
        e0 = base_e + ci * K
        pltpu.make_async_copy(batch_hbm.at[pl.ds(e0, K)], batch_v.at[b],
                              sem_idx.at[b]).wait()
        pltpu.make_async_copy(head_hbm.at[pl.ds(e0, K)], head_v.at[b],
                              sem_idx.at[b]).wait()
        pltpu.make_async_copy(tail_hbm.at[pl.ds(e0, K)], tail_v.at[b],
                              sem_idx.at[b]).wait()
        pltpu.make_async_copy(rel_hbm.at[pl.ds(e0, K), :], rel_v.at[b],
                              sem_rel.at[b]).wait()

    def wait_scatter(b):
        pltpu.make_async_copy(rows_v.at[b], acc_sh.at[tail_sc.at[b]],
                              sem_sc).wait()

    start_load(0, 0)
    pltpu.sync_copy(label_hbm, label_v)
    pltpu.sync_copy(q_hbm, q_v)

    zvec = jnp.zeros((16,), jnp.float32)
    iota16 = lax.iota(jnp.int32, 16)

    def zero_rows(i, _):
        for j in range(D // 16):
            rows_v[0, i, pl.ds(j * 16, 16)] = zvec
        return 0

    lax.fori_loop(0, K, zero_rows, 0)

    def zero_mask(i, _):
        for j in range(D // 16):
            mask_v[i, pl.ds(j * 16, 16)] = zvec
        return 0

    lax.fori_loop(0, MROW, zero_mask, 0)

    def init_iidx(i, _):
        iidx_v[pl.ds(i * 16, 16)] = i * 16 + iota16
        return 0

    lax.fori_loop(0, MROW // 16, init_iidx, 0)

    for k in range(NCHF // NTL + 1):
        off = (s + NTL * k) * K

        @pl.when(off + K <= BN)
        def _():
            pltpu.async_copy(rows_v.at[0],
                             acc_sh.at[pl.ds(pl.multiple_of(off, K), K)],
                             sem_z)

        @pl.when(off == NCHF * K)
        def _():
            pltpu.async_copy(
                rows_v.at[0, pl.ds(0, REM)],
                acc_sh.at[pl.ds(pl.multiple_of(off, 8), REM)], sem_z)
    for k in range(NCHF // NTL + 1):
        off = (s + NTL * k) * K

        @pl.when(off + K <= BN)
        def _():
            pltpu.make_async_copy(
                rows_v.at[0],
                acc_sh.at[pl.ds(pl.multiple_of(off, K), K)], sem_z).wait()

        @pl.when(off == NCHF * K)
        def _():
            pltpu.make_async_copy(
                rows_v.at[0, pl.ds(0, REM)],
                acc_sh.at[pl.ds(pl.multiple_of(off, 8), REM)], sem_z).wait()

    @pl.when(s == 0)
    def _():
        pltpu.sync_copy(mask_v, macc_sh)

    plsc.subcore_barrier()

    cjs = [j * 16 + iota16 for j in range(D // 16)]

    def group16(b, off, tail16):
        head16 = head_v[b, pl.ds(off, 16)]
        batch16 = batch_v[b, pl.ds(off, 16)]
        prior16 = plsc.load_gather(label_v, [head16])
        tail_sc[b, pl.ds(off, 16)] = tail16
        plsc.addupdate_scatter(
            mask_v,
            [lax.shift_right_logical(tail16, 7),
             lax.bitwise_and(tail16, 127)],
            prior16)
        for e in range(0, 16, 8):
            prvs = [prior16.at[jnp.full((16,), e + t, dtype=jnp.int32)]
                    .get(mode='promise_in_bounds') for t in range(8)]
            bss = [batch16[e + t] for t in range(8)]
            for j in range(D // 16):
                rvs = [rel_v[b, off + e + t, pl.ds(j * 16, 16)]
                       for t in range(8)]
                qvs = [q_v[bss[t], pl.ds(j * 16, 16)] for t in range(8)]
                for t in range(8):
                    rows_v[b, off + e + t, pl.ds(j * 16, 16)] = (
                        jnp.maximum(rvs[t] * qvs[t], 0.0) * prvs[t])

    def compute_chunk(b):
        def group_body(g, _):
            off = g * 16
            group16(b, off, tail_v[b, pl.ds(off, 16)])
            return 0

        lax.fori_loop(0, K // 16, group_body, 0)

    def pair_body(p, _):
        for bb in range(2):
            ci = 2 * p + bb
            wait_load(ci, bb)

            @pl.when(ci > 1)
            def _():
                wait_scatter(bb)

            @pl.when(ci + 1 < NCHF)
            def _():
                start_load(ci + 1, 1 - bb)

            compute_chunk(bb)
            pltpu.async_copy(rows_v.at[bb], acc_sh.at[tail_sc.at[bb]],
                             sem_sc, add=True)
        return 0

    lax.fori_loop(0, NCHF // 2, pair_body, 0)
    wait_scatter(0)
    wait_scatter(1)

    e0 = base_e + NCHF * K
    pltpu.sync_copy(batch_hbm.at[pl.ds(e0, REM)],
                    batch_v.at[0, pl.ds(0, REM)])
    pltpu.sync_copy(head_hbm.at[pl.ds(e0, REM)],
                    head_v.at[0, pl.ds(0, REM)])
    pltpu.sync_copy(tail_hbm.at[pl.ds(e0, REM)], tail16_v)
    pltpu.sync_copy(rel_hbm.at[pl.ds(e0, REM), :],
                    rel_v.at[0, pl.ds(0, REM), :])
    group16(0, 0, tail16_v[...])
    pltpu.sync_copy(rows_v.at[0, pl.ds(0, REM)], acc_sh.at[tail16_v],
                    add=True)

    plsc.subcore_barrier()

    pltpu.sync_copy(mask_v, macc_sh.at[iidx_v], add=True)
    plsc.subcore_barrier()
    for k in range(NCHF // NTL + 1):
        off = (s + NTL * k) * K

        @pl.when(off + K <= BN)
        def _():
            aoff = pl.multiple_of(off, K)
            pltpu.async_copy(acc_sh.at[pl.ds(aoff, K)],
                             out_hbm.at[c, pl.ds(aoff, K)], sem_z)

        @pl.when(off == NCHF * K)
        def _():
            aoff = pl.multiple_of(off, 8)
            pltpu.async_copy(acc_sh.at[pl.ds(aoff, REM)],
                             out_hbm.at[c, pl.ds(aoff, REM)], sem_z)
    for k in range(NCHF // NTL + 1):
        off = (s + NTL * k) * K

        @pl.when(off + K <= BN)
        def _():
            aoff = pl.multiple_of(off, K)
            pltpu.make_async_copy(acc_sh.at[pl.ds(aoff, K)],
                                  out_hbm.at[c, pl.ds(aoff, K)], sem_z).wait()

        @pl.when(off == NCHF * K)
        def _():
            aoff = pl.multiple_of(off, 8)
            pltpu.make_async_copy(
                acc_sh.at[pl.ds(aoff, REM)],
                out_hbm.at[c, pl.ds(aoff, REM)], sem_z).wait()

    @pl.when(s == 0)
    def _():
        pltpu.sync_copy(macc_sh, mout_hbm.at[c])


@jax.jit
def _sc_aggregate(rel, q, label, batch_ids, head2edge, tail2edge):
    mesh = plsc.VectorSubcoreMesh(core_axis_name="c", subcore_axis_name="s")
    fn = pl.kernel(
        _sc_aggregate_body,
        out_type=(jax.ShapeDtypeStruct((NSC, BN, D), jnp.float32),
                  jax.ShapeDtypeStruct((NSC, MROW, D), jnp.float32)),
        mesh=mesh,
        scratch_types=[
            pltpu.VMEM((BN,), jnp.float32),
            pltpu.VMEM((B, D), jnp.float32),
            pltpu.VMEM((2, K), jnp.int32),
            pltpu.VMEM((2, K), jnp.int32),
            pltpu.VMEM((2, K), jnp.int32),
            pltpu.VMEM((2, K), jnp.int32),
            pltpu.VMEM((REM,), jnp.int32),
            pltpu.VMEM((2, K, D), jnp.float32),
            pltpu.VMEM((2, K, D), jnp.float32),
            pltpu.VMEM((MROW, D), jnp.float32),
            pltpu.VMEM((MROW,), jnp.int32),
            pltpu.VMEM_SHARED((BN, D), jnp.float32),
            pltpu.VMEM_SHARED((MROW, D), jnp.float32),
            pltpu.SemaphoreType.DMA((2,)),
            pltpu.SemaphoreType.DMA((2,)),
            pltpu.SemaphoreType.DMA,
            pltpu.SemaphoreType.DMA,
        ],
        compiler_params=pltpu.CompilerParams(needs_layout_passes=False),
    )
    return fn(rel, q, label, batch_ids, head2edge, tail2edge)


def _ln(x, g, b, eps=1e-5):
    m = jnp.mean(x, axis=-1, keepdims=True)
    v = jnp.mean((x - m) ** 2, axis=-1, keepdims=True)
    return (x - m) / jnp.sqrt(v + eps) * g + b


def _tc_gru_body(first, last,
                 part_ref, hid_ref, mp_ref, lab_ref, emask_ref,
                 wih_ref, whh_ref, bhh_ref, lng_ref, lnb_ref,
                 wsc_ref, bsc_ref, wffn_ref, bffn_ref,
                 hnew_ref, labnew_ref, *maybe_ffn):
    neighbor = part_ref[0] + part_ref[1]
    g = lng_ref[0]
    bb = lnb_ref[0]
    hid = hid_ref[...]
    if first:
        hid = _ln(hid, g, bb)
    ln1 = _ln(neighbor, g, bb)
    x = lax.dot_general(ln1, wih_ref[...], (((1,), (1,)), ((), ())),
                        preferred_element_type=jnp.float32)
    h = lax.dot_general(hid, whh_ref[...], (((1,), (1,)), ((), ())),
                        preferred_element_type=jnp.float32) + bhh_ref[0]
    u = jax.nn.sigmoid(x[:, :D] + h[:, :D])
    r = jax.nn.sigmoid(x[:, D:2 * D] + h[:, D:2 * D])
    mem = jnp.tanh(x[:, 2 * D:] + r * h[:, 2 * D:])
    hnew = _ln((1.0 - u) * mem + u * hid, g, bb)
    hnew_ref[...] = hnew
    score = (jnp.sum(hnew * wsc_ref[...], axis=1, keepdims=True)
             + bsc_ref[0])
    imask = mp_ref[0] + mp_ref[1]
    m = ((imask + lab_ref[...]) > 1e-8).astype(jnp.float32) * emask_ref[...]
    labnew_ref[...] = jax.nn.sigmoid(m * score + (1.0 - m) * (-1e20))
    if last:
        maybe_ffn[0][...] = lax.dot_general(
            hnew, wffn_ref[...], (((1,), (1,)), ((), ())),
            preferred_element_type=jnp.float32) + bffn_ref[0]


@functools.partial(jax.jit, static_argnums=(0, 1))
def _tc_gru(first, last, part, hid, mflat, lab, emask,
            w_ih, w_hh, b_hh, ln_g, ln_b, w_score, b_score, w_ffn, b_ffn):
    R = 1000
    n_blk = BN // R
    out_shape = [jax.ShapeDtypeStruct((BN, D), jnp.float32),
                 jax.ShapeDtypeStruct((BN, 1), jnp.float32)]
    out_specs = [pl.BlockSpec((R, D), lambda i: (i, 0)),
                 pl.BlockSpec((R, 1), lambda i: (i, 0))]
    if last:
        out_shape.append(jax.ShapeDtypeStruct((BN, D), jnp.float32))
        out_specs.append(pl.BlockSpec((R, D), lambda i: (i, 0)))
    full = lambda shape: pl.BlockSpec(shape, lambda i: tuple(0 for _ in shape))
    return pl.pallas_call(
        functools.partial(_tc_gru_body, first, last),
        grid=(n_blk,),
        in_specs=[
            pl.BlockSpec((NSC, R, D), lambda i: (0, i, 0)),
            pl.BlockSpec((R, D), lambda i: (i, 0)),
            pl.BlockSpec((NSC, R, 1), lambda i: (0, i, 0)),
            pl.BlockSpec((R, 1), lambda i: (i, 0)),
            pl.BlockSpec((R, 1), lambda i: (i, 0)),
            full((NC * D, D)), full((NC * D, D)),
            full((1, NC * D)), full((1, D)), full((1, D)),
            full((1, D)), full((1, 1)), full((D, D)), full((1, D)),
        ],
        out_specs=out_specs,
        out_shape=out_shape,
    )(part, hid, mflat, lab, emask, w_ih, w_hh, b_hh.reshape(1, NC * D),
      ln_g.reshape(1, D), ln_b.reshape(1, D), w_score, b_score.reshape(1, 1),
      w_ffn, b_ffn.reshape(1, D))


def kernel(instructions, entity_emb, fact_relations, topic_label, entity_mask,
           batch_ids, head2edge, tail2edge,
           W_ih, W_hh, b_hh, ln_g, ln_b, W_score, b_score, W_ffn, b_ffn):
    batch_ids = batch_ids.astype(jnp.int32)
    head2edge = head2edge.astype(jnp.int32)
    tail2edge = tail2edge.astype(jnp.int32)
    label = topic_label.reshape(BN)
    emask = entity_mask.reshape(BN, 1)
    hidden = entity_emb.reshape(BN, D)

    labels = []
    ffn_out = None
    for i in range(STEPS):
        part, mpart = _sc_aggregate(fact_relations, instructions[i], label,
                                    batch_ids, head2edge, tail2edge)
        mflat = mpart.reshape(NSC, BNP)[:, :BN].reshape(NSC, BN, 1)
        res = _tc_gru(i == 0, i == STEPS - 1, part, hidden, mflat,
                      label.reshape(BN, 1), emask,
                      W_ih, W_hh, b_hh, ln_g, ln_b, W_score, b_score,
                      W_ffn, b_ffn)
        hidden, labnew = res[0], res[1]
        if i == STEPS - 1:
            ffn_out = res[2]
        label = labnew.reshape(BN)
        labels.append(label.reshape(B, N_LOC))

    return (jnp.stack(labels, axis=0), ffn_out.reshape(B, N_LOC, D))


# speedup vs baseline: 1.0363x; 1.0363x over previous
import functools

import jax
import jax.numpy as jnp
from jax import lax
from jax.experimental import pallas as pl
from jax.experimental.pallas import tpu as pltpu
from jax.experimental.pallas import tpu_sc as plsc

B, N_LOC, D, E, STEPS, NC = 8, 1250, 128, 320000, 3, 3
BN = B * N_LOC

NSC = 2
NTL = 16
NW = NSC * NTL
EPW = E // NW
K = 48
NCHF = EPW // K
REM = EPW - NCHF * K
MROW = 80
BNP = MROW * D


def _sc_aggregate_body(rel_hbm, q_hbm, label_hbm, batch_hbm, head_hbm,
                       tail_hbm, out_hbm, mout_hbm,
                       label_v, q_v, batch_v, head_v, tail_v, tail_sc,
                       tail16_v, rel_v, rows_v, mask_v, iidx_v,
                       acc_sh, macc_sh, sem_rel, sem_idx, sem_sc, sem_z):
    c = lax.axis_index("c")
    s = lax.axis_index("s")
    base_e = (c * NTL + s) * EPW

    def start_load(ci, b):
        e0 = base_e + ci * K
        pltpu.async_copy(batch_hbm.at[pl.ds(e0, K)], batch_v.at[b],
                         sem_idx.at[b])
        pltpu.async_copy(head_hbm.at[pl.ds(e0, K)], head_v.at[b],
                         sem_idx.at[b])
        pltpu.async_copy(tail_hbm.at[pl.ds(e0, K)], tail_v.at[b],
                         sem_idx.at[b])
        pltpu.async_copy(rel_hbm.at[pl.ds(e0, K), :], rel_v.at[b],
                         sem_rel.at[b])

    def wait_load(ci, b):
        e0 = base_e + ci * K
        pltpu.make_async_copy(batch_hbm.at[pl.ds(e0, K)], batch_v.at[b],
                              sem_idx.at[b]).wait()
        pltpu.make_async_copy(head_hbm.at[pl.ds(e0, K)], head_v.at[b],
                              sem_idx.at[b]).wait()
        pltpu.make_async_copy(tail_hbm.at[pl.ds(e0, K)], tail_v.at[b],
                              sem_idx.at[b]).wait()
        pltpu.make_async_copy(rel_hbm.at[pl.ds(e0, K), :], rel_v.at[b],
                              sem_rel.at[b]).wait()

    def wait_scatter(b):
        pltpu.make_async_copy(rows_v.at[b], acc_sh.at[tail_sc.at[b]],
                              sem_sc).wait()

    start_load(0, 0)
    pltpu.sync_copy(label_hbm, label_v)
    pltpu.sync_copy(q_hbm, q_v)

    zvec = jnp.zeros((16,), jnp.float32)
    iota16 = lax.iota(jnp.int32, 16)

    def zero_rows(i, _):
        for j in range(D // 16):
            rows_v[0, i, pl.ds(j * 16, 16)] = zvec
        return 0

    lax.fori_loop(0, K, zero_rows, 0)

    def zero_mask(i, _):
        for j in range(D // 16):
            mask_v[i, pl.ds(j * 16, 16)] = zvec
        return 0

    lax.fori_loop(0, MROW, zero_mask, 0)

    def init_iidx(i, _):
        iidx_v[pl.ds(i * 16, 16)] = i * 16 + iota16
        return 0

    lax.fori_loop(0, MROW // 16, init_iidx, 0)

    for k in range(NCHF // NTL + 1):
        off = (s + NTL * k) * K

        @pl.when(off + K <= BN)
        def _():
            pltpu.async_copy(rows_v.at[0],
                             acc_sh.at[pl.ds(pl.multiple_of(off, K), K)],
                             sem_z)

        @pl.when(off == NCHF * K)
        def _():
            pltpu.async_copy(
                rows_v.at[0, pl.ds(0, REM)],
                acc_sh.at[pl.ds(pl.multiple_of(off, 8), REM)], sem_z)
    for k in range(NCHF // NTL + 1):
        off = (s + NTL * k) * K

        @pl.when(off + K <= BN)
        def _():
            pltpu.make_async_copy(
                rows_v.at[0],
                acc_sh.at[pl.ds(pl.multiple_of(off, K), K)], sem_z).wait()

        @pl.when(off == NCHF * K)
        def _():
            pltpu.make_async_copy(
                rows_v.at[0, pl.ds(0, REM)],
                acc_sh.at[pl.ds(pl.multiple_of(off, 8), REM)], sem_z).wait()

    @pl.when(s == 0)
    def _():
        pltpu.sync_copy(mask_v, macc_sh)

    plsc.subcore_barrier()

    cjs = [j * 16 + iota16 for j in range(D // 16)]

    def group16(b, off, tail16):
        head16 = head_v[b, pl.ds(off, 16)]
        batch16 = batch_v[b, pl.ds(off, 16)]
        prior16 = plsc.load_gather(label_v, [head16])
        tail_sc[b, pl.ds(off, 16)] = tail16
        plsc.addupdate_scatter(
            mask_v,
            [lax.shift_right_logical(tail16, 7),
             lax.bitwise_and(tail16, 127)],
            prior16)
        for e in range(0, 16, 8):
            prvs = [prior16.at[jnp.full((16,), e + t, dtype=jnp.int32)]
                    .get(mode='promise_in_bounds') for t in range(8)]
            bss = [batch16[e + t] for t in range(8)]
            for j in range(D // 16):
                rvs = [rel_v[b, off + e + t, pl.ds(j * 16, 16)]
                       for t in range(8)]
                qvs = [q_v[bss[t], pl.ds(j * 16, 16)] for t in range(8)]
                for t in range(8):
                    rows_v[b, off + e + t, pl.ds(j * 16, 16)] = (
                        jnp.maximum(rvs[t] * qvs[t], 0.0) * prvs[t])

    def compute_chunk(b):
        def group_body(g, _):
            off = g * 16
            group16(b, off, tail_v[b, pl.ds(off, 16)])
            return 0

        lax.fori_loop(0, K // 16, group_body, 0)

    def pair_body(p, _):
        for bb in range(2):
            ci = 2 * p + bb
            wait_load(ci, bb)

            @pl.when(ci > 1)
            def _():
                wait_scatter(bb)

            @pl.when(ci + 1 < NCHF)
            def _():
                start_load(ci + 1, 1 - bb)

            compute_chunk(bb)
            pltpu.async_copy(rows_v.at[bb], acc_sh.at[tail_sc.at[bb]],
                             sem_sc, add=True)
        return 0

    lax.fori_loop(0, NCHF // 2, pair_body, 0)
    wait_scatter(0)
    wait_scatter(1)

    e0 = base_e + NCHF * K
    pltpu.sync_copy(batch_hbm.at[pl.ds(e0, REM)],
                    batch_v.at[0, pl.ds(0, REM)])
    pltpu.sync_copy(head_hbm.at[pl.ds(e0, REM)],
                    head_v.at[0, pl.ds(0, REM)])
    pltpu.sync_copy(tail_hbm.at[pl.ds(e0, REM)], tail16_v)
    pltpu.sync_copy(rel_hbm.at[pl.ds(e0, REM), :],
                    rel_v.at[0, pl.ds(0, REM), :])
    group16(0, 0, tail16_v[...])
    pltpu.sync_copy(rows_v.at[0, pl.ds(0, REM)], acc_sh.at[tail16_v],
                    add=True)

    plsc.subcore_barrier()

    pltpu.sync_copy(mask_v, macc_sh.at[iidx_v], add=True)
    plsc.subcore_barrier()
    for k in range(NCHF // NTL + 1):
        off = (s + NTL * k) * K

        @pl.when(off + K <= BN)
        def _():
            aoff = pl.multiple_of(off, K)
            pltpu.async_copy(acc_sh.at[pl.ds(aoff, K)],
                             out_hbm.at[c, pl.ds(aoff, K)], sem_z)

        @pl.when(off == NCHF * K)
        def _():
            aoff = pl.multiple_of(off, 8)
            pltpu.async_copy(acc_sh.at[pl.ds(aoff, REM)],
                             out_hbm.at[c, pl.ds(aoff, REM)], sem_z)
    for k in range(NCHF // NTL + 1):
        off = (s + NTL * k) * K

        @pl.when(off + K <= BN)
        def _():
            aoff = pl.multiple_of(off, K)
            pltpu.make_async_copy(acc_sh.at[pl.ds(aoff, K)],
                                  out_hbm.at[c, pl.ds(aoff, K)], sem_z).wait()

        @pl.when(off == NCHF * K)
        def _():
            aoff = pl.multiple_of(off, 8)
            pltpu.make_async_copy(
                acc_sh.at[pl.ds(aoff, REM)],
                out_hbm.at[c, pl.ds(aoff, REM)], sem_z).wait()

    @pl.when(s == 0)
    def _():
        pltpu.sync_copy(macc_sh, mout_hbm.at[c])


@jax.jit
def _sc_aggregate(rel, q, label, batch_ids, head2edge, tail2edge):
    mesh = plsc.VectorSubcoreMesh(core_axis_name="c", subcore_axis_name="s")
    fn = pl.kernel(
        _sc_aggregate_body,
        out_type=(jax.ShapeDtypeStruct((NSC, BN, D), jnp.float32),
                  jax.ShapeDtypeStruct((NSC, MROW, D), jnp.float32)),
        mesh=mesh,
        scratch_types=[
            pltpu.VMEM((BN,), jnp.float32),
            pltpu.VMEM((B, D), jnp.float32),
            pltpu.VMEM((2, K), jnp.int32),
            pltpu.VMEM((2, K), jnp.int32),
            pltpu.VMEM((2, K), jnp.int32),
            pltpu.VMEM((2, K), jnp.int32),
            pltpu.VMEM((REM,), jnp.int32),
            pltpu.VMEM((2, K, D), jnp.float32),
            pltpu.VMEM((2, K, D), jnp.float32),
            pltpu.VMEM((MROW, D), jnp.float32),
            pltpu.VMEM((MROW,), jnp.int32),
            pltpu.VMEM_SHARED((BN, D), jnp.float32),
            pltpu.VMEM_SHARED((MROW, D), jnp.float32),
            pltpu.SemaphoreType.DMA((2,)),
            pltpu.SemaphoreType.DMA((2,)),
            pltpu.SemaphoreType.DMA,
            pltpu.SemaphoreType.DMA,
        ],
        compiler_params=pltpu.CompilerParams(needs_layout_passes=False),
    )
    return fn(rel, q, label, batch_ids, head2edge, tail2edge)


def _ln(x, g, b, eps=1e-5):
    m = jnp.mean(x, axis=-1, keepdims=True)
    v = jnp.mean((x - m) ** 2, axis=-1, keepdims=True)
    return (x - m) / jnp.sqrt(v + eps) * g + b


def _tc_gru_body(first, last,
                 part_ref, hid_ref,
                 wih_ref, whh_ref, bhh_ref, lng_ref, lnb_ref,
                 wsc_ref, bsc_ref, wffn_ref, bffn_ref,
                 hnew_ref, score_ref, *maybe_ffn):
    neighbor = part_ref[0] + part_ref[1]
    g = lng_ref[0]
    bb = lnb_ref[0]
    hid = hid_ref[...]
    if first:
        hid = _ln(hid, g, bb)
    ln1 = _ln(neighbor, g, bb)
    x = lax.dot_general(ln1, wih_ref[...], (((1,), (1,)), ((), ())),
                        preferred_element_type=jnp.float32)
    h = lax.dot_general(hid, whh_ref[...], (((1,), (1,)), ((), ())),
                        preferred_element_type=jnp.float32) + bhh_ref[0]
    u = jax.nn.sigmoid(x[:, :D] + h[:, :D])
    r = jax.nn.sigmoid(x[:, D:2 * D] + h[:, D:2 * D])
    mem = jnp.tanh(x[:, 2 * D:] + r * h[:, 2 * D:])
    hnew = _ln((1.0 - u) * mem + u * hid, g, bb)
    hnew_ref[...] = hnew
    score_ref[...] = (jnp.sum(hnew * wsc_ref[...], axis=1, keepdims=True)
                      + bsc_ref[0])
    if last:
        maybe_ffn[0][...] = lax.dot_general(
            hnew, wffn_ref[...], (((1,), (1,)), ((), ())),
            preferred_element_type=jnp.float32) + bffn_ref[0]


@functools.partial(jax.jit, static_argnums=(0, 1))
def _tc_gru(first, last, part, hid,
            w_ih, w_hh, b_hh, ln_g, ln_b, w_score, b_score, w_ffn, b_ffn):
    R = 1000
    n_blk = BN // R
    out_shape = [jax.ShapeDtypeStruct((BN, D), jnp.float32),
                 jax.ShapeDtypeStruct((BN, 1), jnp.float32)]
    out_specs = [pl.BlockSpec((R, D), lambda i: (i, 0)),
                 pl.BlockSpec((R, 1), lambda i: (i, 0))]
    if last:
        out_shape.append(jax.ShapeDtypeStruct((BN, D), jnp.float32))
        out_specs.append(pl.BlockSpec((R, D), lambda i: (i, 0)))
    full = lambda shape: pl.BlockSpec(shape, lambda i: tuple(0 for _ in shape))
    return pl.pallas_call(
        functools.partial(_tc_gru_body, first, last),
        grid=(n_blk,),
        in_specs=[
            pl.BlockSpec((NSC, R, D), lambda i: (0, i, 0)),
            pl.BlockSpec((R, D), lambda i: (i, 0)),
            full((NC * D, D)), full((NC * D, D)),
            full((1, NC * D)), full((1, D)), full((1, D)),
            full((1, D)), full((1, 1)), full((D, D)), full((1, D)),
        ],
        out_specs=out_specs,
        out_shape=out_shape,
    )(part, hid, w_ih, w_hh, b_hh.reshape(1, NC * D),
      ln_g.reshape(1, D), ln_b.reshape(1, D), w_score, b_score.reshape(1, 1),
      w_ffn, b_ffn.reshape(1, D))


def _tc_label_body(mpart_ref, score_ref, lab_ref, emask_ref, out_ref):
    imask = mpart_ref[0] + mpart_ref[1]
    m = ((imask + lab_ref[...]) > 1e-8).astype(jnp.float32) * emask_ref[...]
    out_ref[...] = jax.nn.sigmoid(
        m * score_ref[...] + (1.0 - m) * (-1e20))


@jax.jit
def _tc_label(mpart, score2d, lab2d, emask2d):
    return pl.pallas_call(
        _tc_label_body,
        out_shape=jax.ShapeDtypeStruct((MROW, D), jnp.float32),
    )(mpart, score2d, lab2d, emask2d)


def _to2d(x_flat):
    return jnp.concatenate(
        [x_flat, jnp.zeros((BNP - BN,), jnp.float32)]).reshape(MROW, D)


def kernel(instructions, entity_emb, fact_relations, topic_label, entity_mask,
           batch_ids, head2edge, tail2edge,
           W_ih, W_hh, b_hh, ln_g, ln_b, W_score, b_score, W_ffn, b_ffn):
    batch_ids = batch_ids.astype(jnp.int32)
    head2edge = head2edge.astype(jnp.int32)
    tail2edge = tail2edge.astype(jnp.int32)
    label = topic_label.reshape(BN)
    emask2d = _to2d(entity_mask.reshape(BN))
    hidden = entity_emb.reshape(BN, D)

    labels = []
    ffn_out = None
    for i in range(STEPS):
        part, mpart = _sc_aggregate(fact_relations, instructions[i], label,
                                    batch_ids, head2edge, tail2edge)
        res = _tc_gru(i == 0, i == STEPS - 1, part, hidden,
                      W_ih, W_hh, b_hh, ln_g, ln_b, W_score, b_score,
                      W_ffn, b_ffn)
        hidden, score = res[0], res[1]
        if i == STEPS - 1:
            ffn_out = res[2]
        lab2d = _tc_label(mpart, _to2d(score.reshape(BN)), _to2d(label),
                          emask2d)
        label = lab2d.reshape(BNP)[:BN]
        labels.append(label.reshape(B, N_LOC))

    return (jnp.stack(labels, axis=0), ffn_out.reshape(B, N_LOC, D))
